# Initial kernel scaffold; baseline (speedup 1.0000x reference)
import jax
import jax.numpy as jnp
from jax.experimental import pallas as pl


def kernel(x, edge_index, weight):
    raise NotImplementedError("write your pallas kernel here")


# speedup vs baseline: 14.4740x; 14.4740x over previous
import functools

import jax
import jax.numpy as jnp
from jax import lax
from jax.experimental import pallas as pl
from jax.experimental.pallas import tpu as pltpu
from jax.experimental.pallas import tpu_sc as plsc

EPS = 1e-05
T_ITERS = 4
NC = 2
NS = 16
NW = NC * NS
L = 16


def _oni_norm(w):
    d = w.shape[0]
    zc = w - jnp.mean(w, axis=1, keepdims=True)
    s = lax.dot_general(zc, zc, (((1,), (1,)), ((), ())),
                        preferred_element_type=jnp.float32)
    eye = jnp.eye(d, dtype=jnp.float32)
    s = s + EPS * eye
    norm_s = jnp.sqrt(jnp.sum(s * s))
    s = s / norm_s
    b = eye
    for _ in range(T_ITERS):
        b2 = jnp.dot(b, b, preferred_element_type=jnp.float32)
        b3 = jnp.dot(b2, b, preferred_element_type=jnp.float32)
        b = 1.5 * b - 0.5 * jnp.dot(b3, s, preferred_element_type=jnp.float32)
    return jnp.dot(b, zc, preferred_element_type=jnp.float32) / jnp.sqrt(norm_s)


def _make_sc_kernels(n_pad, d, epw, ch):
    mesh = plsc.VectorSubcoreMesh(core_axis_name="c", subcore_axis_name="s")
    rpt = n_pad // NS
    nch = epw // ch

    @functools.partial(
        pl.kernel,
        out_type=jax.ShapeDtypeStruct((NW, n_pad), jnp.float32),
        mesh=mesh,
        scratch_types=[
            pltpu.VMEM((n_pad,), jnp.float32),
            pltpu.VMEM((epw,), jnp.int32),
        ],
        compiler_params=pltpu.CompilerParams(needs_layout_passes=False),
    )
    def hist_kernel(col_hbm, out_hbm, hist_v, idx_v):
        cid = lax.axis_index("c")
        sid = lax.axis_index("s")
        wid = sid * NC + cid
        zeros16 = jnp.zeros((L,), jnp.float32)

        def zbody(i, _):
            hist_v[pl.ds(i * L, L)] = zeros16
            return 0
        lax.fori_loop(0, n_pad // L, zbody, 0)

        pltpu.sync_copy(col_hbm.at[pl.ds(wid * epw, epw)], idx_v)
        ones16 = jnp.ones((L,), jnp.float32)

        def body(j, _):
            idx = idx_v[pl.ds(j * L, L)]
            plsc.addupdate_scatter(hist_v, [idx], ones16)
            return 0
        lax.fori_loop(0, epw // L, body, 0)
        pltpu.sync_copy(hist_v, out_hbm.at[wid])

    @functools.partial(
        pl.kernel,
        out_type=jax.ShapeDtypeStruct((NC, n_pad, d), jnp.float32),
        mesh=mesh,
        scratch_types=[
            pltpu.VMEM_SHARED((n_pad, d), jnp.float32),
            pltpu.VMEM((ch,), jnp.int32),
            pltpu.VMEM((ch,), jnp.int32),
            pltpu.VMEM((ch, d), jnp.float32),
            pltpu.SemaphoreType.DMA,
        ],
    )
    def agg_kernel(xs_hbm, row_hbm, col_hbm, out_hbm,
                   acc_sh, ridx_v, cidx_v, gbuf_v, sem):
        cid = lax.axis_index("c")
        sid = lax.axis_index("s")
        wid = sid * NC + cid
        rbase = sid * rpt
        pltpu.sync_copy(xs_hbm.at[pl.ds(rbase, rpt)],
                        acc_sh.at[pl.ds(rbase, rpt)])
        plsc.subcore_barrier()

        ebase = wid * epw

        def body(k, _):
            off = ebase + k * ch
            pltpu.sync_copy(row_hbm.at[pl.ds(off, ch)], ridx_v)
            pltpu.sync_copy(col_hbm.at[pl.ds(off, ch)], cidx_v)
            pltpu.async_copy(xs_hbm.at[ridx_v], gbuf_v, sem).wait()
            pltpu.sync_copy(gbuf_v, acc_sh.at[cidx_v], add=True)
            return 0
        lax.fori_loop(0, nch, body, 0)

        plsc.subcore_barrier()
        pltpu.sync_copy(acc_sh.at[pl.ds(rbase, rpt)],
                        out_hbm.at[cid, pl.ds(rbase, rpt)])

    return hist_kernel, agg_kernel


def kernel(x, edge_index, weight):
    n, d = x.shape
    e = edge_index.shape[1]

    blk = 640
    n_pad = ((n + blk - 1) // blk) * blk
    ch = 128
    epw = ((e + NW * ch - 1) // (NW * ch)) * ch
    e_pad = epw * NW

    row = jnp.concatenate(
        [edge_index[0], jnp.zeros((e_pad - e,), jnp.int32)])
    col = jnp.concatenate(
        [edge_index[1], jnp.full((e_pad - e,), n, jnp.int32)])
    x_pad = jnp.pad(x, ((0, n_pad - n), (0, 0)))

    hist_kernel, agg_kernel = _make_sc_kernels(n_pad, d, epw, ch)
    hist = hist_kernel(col)

    grid_a = n_pad // blk

    def tc_a_body(hist_ref, x_ref, w_ref, xs_ref):
        deg = jnp.sum(hist_ref[...], axis=0) + 1.0
        dis = lax.rsqrt(deg)
        w = _oni_norm(w_ref[...])
        xs_ref[...] = dis[:, None] * jnp.dot(
            x_ref[...], w, preferred_element_type=jnp.float32)

    xs = pl.pallas_call(
        tc_a_body,
        grid=(grid_a,),
        in_specs=[
            pl.BlockSpec((NW, blk), lambda i: (0, i)),
            pl.BlockSpec((blk, d), lambda i: (i, 0)),
            pl.BlockSpec((d, d), lambda i: (0, 0)),
        ],
        out_specs=pl.BlockSpec((blk, d), lambda i: (i, 0)),
        out_shape=jax.ShapeDtypeStruct((n_pad, d), jnp.float32),
    )(hist, x_pad, weight)

    parts = agg_kernel(xs, row, col)

    blkf = 640
    grid_f = (n + blkf - 1) // blkf

    def tc_f_body(hist_ref, p_ref, xs_ref, out_ref):
        deg = jnp.sum(hist_ref[...], axis=0) + 1.0
        dis = lax.rsqrt(deg)
        out_ref[...] = dis[:, None] * (p_ref[0] + p_ref[1] - xs_ref[...])

    out = pl.pallas_call(
        tc_f_body,
        grid=(grid_f,),
        in_specs=[
            pl.BlockSpec((NW, blkf), lambda i: (0, i)),
            pl.BlockSpec((NC, blkf, d), lambda i: (0, i, 0)),
            pl.BlockSpec((blkf, d), lambda i: (i, 0)),
        ],
        out_specs=pl.BlockSpec((blkf, d), lambda i: (i, 0)),
        out_shape=jax.ShapeDtypeStruct((n, d), jnp.float32),
    )(hist, parts, xs)
    return out
